# speedup vs baseline: 1.8962x; 1.8962x over previous
import jax
import jax.numpy as jnp
from jax.experimental import pallas as pl

_BLOCK_ROWS = 512


def _softmax_block(logits_ref, out_ref):
    v = logits_ref[...]
    m = jnp.max(v, axis=1, keepdims=True)
    e = jnp.exp(v - m)
    s = jnp.sum(e, axis=1, keepdims=True)
    out_ref[...] = e / s


def kernel(x, logits):
    del x
    n_rows, n_cols = logits.shape
    grid = (n_rows // _BLOCK_ROWS,)
    return pl.pallas_call(
        _softmax_block,
        grid=grid,
        in_specs=[pl.BlockSpec((_BLOCK_ROWS, n_cols), lambda i: (i, 0))],
        out_specs=pl.BlockSpec((_BLOCK_ROWS, n_cols), lambda i: (i, 0)),
        out_shape=jax.ShapeDtypeStruct((n_rows, n_cols), logits.dtype),
    )(logits)
